# baseline (speedup 1.0000x reference)
import functools

import jax
import jax.numpy as jnp
from jax import lax
from jax.experimental import pallas as pl
from jax.experimental.pallas import tpu as pltpu
from jax.experimental.pallas import tpu_sc as plsc

E = 8
T = 2048
D = 1024
LANES = 128
TM = 128
NT = 24
P = NT * TM
NC = 2
NS = 16
NW = NC * NS


def _routing_body(x_ref, gW_ref, gb_ref, sid_ref, dest_ref, te_ref):
    f32 = jnp.float32
    i32 = jnp.int32
    x = x_ref[...]
    gate = jnp.dot(x, gW_ref[...], preferred_element_type=f32) + gb_ref[...]
    mx = jnp.max(gate, axis=1, keepdims=True)
    lane = lax.broadcasted_iota(i32, (T, LANES), 1)
    idx = jnp.min(jnp.where(gate == mx, lane, LANES), axis=1, keepdims=True)
    m = (lane == idx).astype(f32)

    CH = 256
    rch = lax.broadcasted_iota(i32, (CH, CH), 0)
    cch = lax.broadcasted_iota(i32, (CH, CH), 1)
    L = (rch >= cch).astype(f32)
    chunks = []
    carry = jnp.zeros((1, LANES), f32)
    for ci in range(T // CH):
        cs = jnp.dot(L, m[ci * CH:(ci + 1) * CH, :],
                     preferred_element_type=f32) + carry
        carry = cs[CH - 1:CH, :]
        chunks.append(cs)
    csum = jnp.concatenate(chunks, axis=0)

    rank = jnp.sum(csum * m, axis=1, keepdims=True) - 1.0
    counts_row = csum[T - 1:T, :]
    nt_row = (counts_row.astype(i32) + (TM - 1)) >> 7
    pc_row = (nt_row << 7).astype(f32)

    r2 = lax.broadcasted_iota(i32, (LANES, LANES), 0)
    c2 = lax.broadcasted_iota(i32, (LANES, LANES), 1)
    SU = (r2 < c2).astype(f32)
    base_row = jnp.dot(pc_row, SU, preferred_element_type=f32)
    base_tok = jnp.sum(base_row * m, axis=1, keepdims=True)
    dest = (base_tok + rank).astype(i32)
    dest_ref[...] = dest

    SL = (r2 > c2).astype(f32)
    counts_col = lax.dot_general(
        m, jnp.ones((T, 1), f32), (((0,), (0,)), ((), ())),
        preferred_element_type=f32)
    nt_col = (counts_col.astype(i32) + (TM - 1)) >> 7
    pc_col = (nt_col << 7).astype(f32)
    base_col = jnp.dot(SL, pc_col, preferred_element_type=f32).astype(i32)
    cmp = (base_col <= c2 * TM).astype(i32)
    te = jnp.clip(jnp.sum(cmp, axis=0, keepdims=True) - 1, 0, E - 1)
    total_tiles = jnp.sum(nt_row, axis=1, keepdims=True)
    iotaL = lax.broadcasted_iota(i32, (1, LANES), 1)
    te_ref[...] = jnp.where(iotaL < total_tiles, te, -1)

    q = dest >> 7
    rr = dest & (TM - 1)
    OH = (rr == lane).astype(f32)
    tvec = lax.broadcasted_iota(i32, (T, 1), 0).astype(f32)
    Bmask = (q == lane).astype(f32)
    B = Bmask * tvec
    S = lax.dot_general(B, OH, (((0,), (0,)), ((), ())),
                        preferred_element_type=f32,
                        precision=lax.Precision.HIGHEST)
    V = lax.dot_general(Bmask, OH, (((0,), (0,)), ((), ())),
                        preferred_element_type=f32)
    pidx = (((r2 << 7) + c2) & (T - 1)).astype(f32)
    sid_ref[...] = (S + (1.0 - V) * pidx).astype(i32)


def _routing(x, gate_W, gate_b):
    gWp = jnp.zeros((D, LANES), jnp.float32).at[:, :E].set(gate_W)
    gbp = jnp.full((1, LANES), -1e30, jnp.float32).at[0, :E].set(gate_b)
    return pl.pallas_call(
        _routing_body,
        out_shape=[
            jax.ShapeDtypeStruct((LANES, LANES), jnp.int32),
            jax.ShapeDtypeStruct((T, 1), jnp.int32),
            jax.ShapeDtypeStruct((1, LANES), jnp.int32),
        ],
    )(x, gWp, gbp)


def _sc_gather(table, ids):
    n = ids.shape[0]
    rows_w = n // NW
    mesh = plsc.VectorSubcoreMesh(core_axis_name="c", subcore_axis_name="s")

    @functools.partial(
        pl.kernel,
        out_type=jax.ShapeDtypeStruct((n, D), jnp.float32),
        mesh=mesh,
        scratch_types=[
            pltpu.VMEM((rows_w,), jnp.int32),
            pltpu.VMEM((rows_w, D), jnp.float32),
            pltpu.SemaphoreType.DMA,
        ],
    )
    def k(table_hbm, idx_hbm, out_hbm, idx_v, rows_v, sem):
        wid = lax.axis_index("s") * NC + lax.axis_index("c")
        base = wid * rows_w
        pltpu.sync_copy(idx_hbm.at[pl.ds(base, rows_w)], idx_v)
        pltpu.async_copy(table_hbm.at[idx_v], rows_v, sem).wait()
        pltpu.sync_copy(rows_v, out_hbm.at[pl.ds(base, rows_w)])

    return k(table, ids)


def _gmm_body(te_ref, xs_ref, W_hbm, b_ref, out_ref, W_vmem, sem):
    i = pl.program_id(0)

    @pl.when(i == 0)
    def _():
        for e in range(E):
            pltpu.make_async_copy(W_hbm.at[e], W_vmem.at[e], sem).start()
        for e in range(E):
            pltpu.make_async_copy(W_hbm.at[e], W_vmem.at[e], sem).wait()

    te = te_ref[0, i]

    @pl.when(te >= 0)
    def _():
        out_ref[...] = lax.dot_general(
            xs_ref[...].astype(jnp.bfloat16), W_vmem[te],
            (((1,), (1,)), ((), ())),
            preferred_element_type=jnp.float32,
        ) + b_ref[te]


def _grouped_matmul(x_sorted, expert_W, expert_b, te):
    grid_spec = pltpu.PrefetchScalarGridSpec(
        num_scalar_prefetch=1,
        grid=(NT,),
        in_specs=[
            pl.BlockSpec((TM, D), lambda i, te: (i, 0)),
            pl.BlockSpec(memory_space=pltpu.MemorySpace.HBM),
            pl.BlockSpec((E, 1, D), lambda i, te: (0, 0, 0)),
        ],
        out_specs=pl.BlockSpec((TM, D), lambda i, te: (i, 0)),
        scratch_shapes=[
            pltpu.VMEM((E, D, D), jnp.bfloat16),
            pltpu.SemaphoreType.DMA,
        ],
    )
    return pl.pallas_call(
        _gmm_body,
        grid_spec=grid_spec,
        out_shape=jax.ShapeDtypeStruct((P, D), jnp.float32),
        compiler_params=pltpu.CompilerParams(
            dimension_semantics=("arbitrary",),
        ),
    )(te, x_sorted, expert_W.astype(jnp.bfloat16), expert_b.reshape(E, 1, D))


def kernel(x, expert_W, expert_b, gate_W, gate_b):
    sid, dest, te = _routing(x, gate_W, gate_b)
    sids = sid[:NT].reshape(P)
    x_sorted = _sc_gather(x, sids)
    out_sorted = _grouped_matmul(x_sorted, expert_W, expert_b, te)
    return _sc_gather(out_sorted, dest.reshape(T))


# speedup vs baseline: 1.1136x; 1.1136x over previous
import functools

import jax
import jax.numpy as jnp
from jax import lax
from jax.experimental import pallas as pl
from jax.experimental.pallas import tpu as pltpu
from jax.experimental.pallas import tpu_sc as plsc

E = 8
T = 2048
D = 1024
LANES = 128
TM = 128
NT = 24
P = NT * TM
NC = 2
NS = 16
NW = NC * NS


def _routing_body(x_ref, gW_ref, gb_ref, sid_ref, dest_ref, te_ref):
    f32 = jnp.float32
    i32 = jnp.int32
    x = x_ref[...]
    gate = jnp.dot(x, gW_ref[...], preferred_element_type=f32) + gb_ref[...]
    mx = jnp.max(gate, axis=1, keepdims=True)
    lane = lax.broadcasted_iota(i32, (T, LANES), 1)
    idx = jnp.min(jnp.where(gate == mx, lane, LANES), axis=1, keepdims=True)
    m = (lane == idx).astype(f32)

    CH = 256
    rch = lax.broadcasted_iota(i32, (CH, CH), 0)
    cch = lax.broadcasted_iota(i32, (CH, CH), 1)
    L = (rch >= cch).astype(f32)
    chunks = []
    carry = jnp.zeros((1, LANES), f32)
    for ci in range(T // CH):
        cs = jnp.dot(L, m[ci * CH:(ci + 1) * CH, :],
                     preferred_element_type=f32) + carry
        carry = cs[CH - 1:CH, :]
        chunks.append(cs)
    csum = jnp.concatenate(chunks, axis=0)

    rank = jnp.sum(csum * m, axis=1, keepdims=True) - 1.0
    counts_row = csum[T - 1:T, :]
    nt_row = (counts_row.astype(i32) + (TM - 1)) >> 7
    pc_row = (nt_row << 7).astype(f32)

    r2 = lax.broadcasted_iota(i32, (LANES, LANES), 0)
    c2 = lax.broadcasted_iota(i32, (LANES, LANES), 1)
    SU = (r2 < c2).astype(f32)
    base_row = jnp.dot(pc_row, SU, preferred_element_type=f32)
    base_tok = jnp.sum(base_row * m, axis=1, keepdims=True)
    dest = (base_tok + rank).astype(i32)
    dest_ref[...] = dest

    SL = (r2 > c2).astype(f32)
    counts_col = lax.dot_general(
        m, jnp.ones((T, 1), f32), (((0,), (0,)), ((), ())),
        preferred_element_type=f32)
    nt_col = (counts_col.astype(i32) + (TM - 1)) >> 7
    pc_col = (nt_col << 7).astype(f32)
    base_col = jnp.dot(SL, pc_col, preferred_element_type=f32).astype(i32)
    cmp = (base_col <= c2 * TM).astype(i32)
    te = jnp.clip(jnp.sum(cmp, axis=0, keepdims=True) - 1, 0, E - 1)
    total_tiles = jnp.sum(nt_row, axis=1, keepdims=True)
    iotaL = lax.broadcasted_iota(i32, (1, LANES), 1)
    te_ref[...] = jnp.where(iotaL < total_tiles, te, -1)

    q = dest >> 7
    rr = dest & (TM - 1)
    OH = (rr == lane).astype(f32)
    tvec = lax.broadcasted_iota(i32, (T, 1), 0).astype(f32)
    Bmask = (q == lane).astype(f32)
    B = Bmask * tvec
    S = lax.dot_general(B, OH, (((0,), (0,)), ((), ())),
                        preferred_element_type=f32,
                        precision=lax.Precision.HIGHEST)
    V = lax.dot_general(Bmask, OH, (((0,), (0,)), ((), ())),
                        preferred_element_type=f32)
    pidx = (((r2 << 7) + c2) & (T - 1)).astype(f32)
    sid_ref[...] = (S + (1.0 - V) * pidx).astype(i32)


def _routing(x, gate_W, gate_b):
    gWp = jnp.zeros((D, LANES), jnp.float32).at[:, :E].set(gate_W)
    gbp = jnp.full((1, LANES), -1e30, jnp.float32).at[0, :E].set(gate_b)
    return pl.pallas_call(
        _routing_body,
        out_shape=[
            jax.ShapeDtypeStruct((LANES, LANES), jnp.int32),
            jax.ShapeDtypeStruct((T, 1), jnp.int32),
            jax.ShapeDtypeStruct((1, LANES), jnp.int32),
        ],
    )(x, gWp, gbp)


def _sc_gather(table, ids):
    n = ids.shape[0]
    rows_w = n // NW
    mesh = plsc.VectorSubcoreMesh(core_axis_name="c", subcore_axis_name="s")

    @functools.partial(
        pl.kernel,
        out_type=jax.ShapeDtypeStruct((n, D), jnp.float32),
        mesh=mesh,
        scratch_types=[
            pltpu.VMEM((rows_w,), jnp.int32),
            pltpu.VMEM((rows_w, D), jnp.float32),
            pltpu.SemaphoreType.DMA,
        ],
    )
    def k(table_hbm, idx_hbm, out_hbm, idx_v, rows_v, sem):
        wid = lax.axis_index("s") * NC + lax.axis_index("c")
        base = wid * rows_w
        pltpu.sync_copy(idx_hbm.at[pl.ds(base, rows_w)], idx_v)
        pltpu.async_copy(table_hbm.at[idx_v], rows_v, sem).wait()
        pltpu.sync_copy(rows_v, out_hbm.at[pl.ds(base, rows_w)])

    return k(table, ids)


def _gmm_body(te_ref, xs_ref, W_hbm, b_ref, out_ref, W_vmem, sem):
    i = pl.program_id(0)

    @pl.when(i == 0)
    def _():
        for e in range(E):
            pltpu.make_async_copy(W_hbm.at[e], W_vmem.at[e], sem).start()
        for e in range(E):
            pltpu.make_async_copy(W_hbm.at[e], W_vmem.at[e], sem).wait()

    te = jnp.maximum(te_ref[0, i], 0)
    out_ref[...] = lax.dot_general(
        xs_ref[...], W_vmem[te],
        (((1,), (1,)), ((), ())),
        preferred_element_type=jnp.float32,
    ) + b_ref[te]


def _grouped_matmul(x_sorted, expert_W, expert_b, te):
    grid_spec = pltpu.PrefetchScalarGridSpec(
        num_scalar_prefetch=1,
        grid=(NT,),
        in_specs=[
            pl.BlockSpec((TM, D), lambda i, te: (i, 0)),
            pl.BlockSpec(memory_space=pltpu.MemorySpace.HBM),
            pl.BlockSpec((E, 1, D), lambda i, te: (0, 0, 0)),
        ],
        out_specs=pl.BlockSpec((TM, D), lambda i, te: (i, 0)),
        scratch_shapes=[
            pltpu.VMEM((E, D, D), jnp.float32),
            pltpu.SemaphoreType.DMA,
        ],
    )
    return pl.pallas_call(
        _gmm_body,
        grid_spec=grid_spec,
        out_shape=jax.ShapeDtypeStruct((P, D), jnp.float32),
        compiler_params=pltpu.CompilerParams(
            dimension_semantics=("parallel",),
        ),
    )(te, x_sorted, expert_W, expert_b.reshape(E, 1, D))


def kernel(x, expert_W, expert_b, gate_W, gate_b):
    sid, dest, te = _routing(x, gate_W, gate_b)
    sids = sid[:NT].reshape(P)
    x_sorted = _sc_gather(x, sids)
    out_sorted = _grouped_matmul(x_sorted, expert_W, expert_b, te)
    return _sc_gather(out_sorted, dest.reshape(T))


# speedup vs baseline: 1.1311x; 1.0156x over previous
import functools

import jax
import jax.numpy as jnp
from jax import lax
from jax.experimental import pallas as pl
from jax.experimental.pallas import tpu as pltpu
from jax.experimental.pallas import tpu_sc as plsc

E = 8
T = 2048
D = 1024
LANES = 128
TM = 128
NT = 24
P = NT * TM
NC = 2
NS = 16
NW = NC * NS


def _routing_body(x_ref, gW_ref, gb_ref, sid_ref, dest_ref, te_ref):
    f32 = jnp.float32
    i32 = jnp.int32
    x = x_ref[...]
    gate = jnp.dot(x, gW_ref[...], preferred_element_type=f32) + gb_ref[...]
    mx = jnp.max(gate, axis=1, keepdims=True)
    lane = lax.broadcasted_iota(i32, (T, LANES), 1)
    idx = jnp.min(jnp.where(gate == mx, lane, LANES), axis=1, keepdims=True)
    m = (lane == idx).astype(f32)

    CH = 256
    rch = lax.broadcasted_iota(i32, (CH, CH), 0)
    cch = lax.broadcasted_iota(i32, (CH, CH), 1)
    L = (rch >= cch).astype(f32)
    chunks = []
    carry = jnp.zeros((1, LANES), f32)
    for ci in range(T // CH):
        cs = jnp.dot(L, m[ci * CH:(ci + 1) * CH, :],
                     preferred_element_type=f32) + carry
        carry = cs[CH - 1:CH, :]
        chunks.append(cs)
    csum = jnp.concatenate(chunks, axis=0)

    rank = jnp.sum(csum * m, axis=1, keepdims=True) - 1.0
    counts_row = csum[T - 1:T, :]
    nt_row = (counts_row.astype(i32) + (TM - 1)) >> 7
    pc_row = (nt_row << 7).astype(f32)

    r2 = lax.broadcasted_iota(i32, (LANES, LANES), 0)
    c2 = lax.broadcasted_iota(i32, (LANES, LANES), 1)
    SU = (r2 < c2).astype(f32)
    base_row = jnp.dot(pc_row, SU, preferred_element_type=f32)
    base_tok = jnp.sum(base_row * m, axis=1, keepdims=True)
    dest = (base_tok + rank).astype(i32)
    dest_ref[...] = dest

    SL = (r2 > c2).astype(f32)
    counts_col = lax.dot_general(
        m, jnp.ones((T, 1), f32), (((0,), (0,)), ((), ())),
        preferred_element_type=f32)
    nt_col = (counts_col.astype(i32) + (TM - 1)) >> 7
    pc_col = (nt_col << 7).astype(f32)
    base_col = jnp.dot(SL, pc_col, preferred_element_type=f32).astype(i32)
    cmp = (base_col <= c2 * TM).astype(i32)
    te = jnp.clip(jnp.sum(cmp, axis=0, keepdims=True) - 1, 0, E - 1)
    total_tiles = jnp.sum(nt_row, axis=1, keepdims=True)
    iotaL = lax.broadcasted_iota(i32, (1, LANES), 1)
    te_last = jnp.max(jnp.where(counts_row > 0.0, iotaL, -1),
                      axis=1, keepdims=True)
    te_full = jnp.where(iotaL < total_tiles, te, te_last)
    SH = ((c2 == r2 + NT + E) & (r2 < E)).astype(f32)
    nt_shift = jnp.dot(nt_row.astype(f32), SH,
                       preferred_element_type=f32).astype(i32)
    te_ref[...] = jnp.where(iotaL < NT, te_full, nt_shift)

    q = dest >> 7
    rr = dest & (TM - 1)
    OH = (rr == lane).astype(f32)
    tvec = lax.broadcasted_iota(i32, (T, 1), 0).astype(f32)
    Bmask = (q == lane).astype(f32)
    B = Bmask * tvec
    S = lax.dot_general(B, OH, (((0,), (0,)), ((), ())),
                        preferred_element_type=f32,
                        precision=lax.Precision.HIGHEST)
    V = lax.dot_general(Bmask, OH, (((0,), (0,)), ((), ())),
                        preferred_element_type=f32)
    pidx = (((r2 << 7) + c2) & (T - 1)).astype(f32)
    sid_ref[...] = (S + (1.0 - V) * pidx).astype(i32)


def _routing(x, gate_W, gate_b):
    gWp = jnp.zeros((D, LANES), jnp.float32).at[:, :E].set(gate_W)
    gbp = jnp.full((1, LANES), -1e30, jnp.float32).at[0, :E].set(gate_b)
    return pl.pallas_call(
        _routing_body,
        out_shape=[
            jax.ShapeDtypeStruct((LANES, LANES), jnp.int32),
            jax.ShapeDtypeStruct((T, 1), jnp.int32),
            jax.ShapeDtypeStruct((1, LANES), jnp.int32),
        ],
    )(x, gWp, gbp)


def _sc_gather(table, ids):
    n = ids.shape[0]
    rows_w = n // NW
    dt = table.dtype
    mesh = plsc.VectorSubcoreMesh(core_axis_name="c", subcore_axis_name="s")

    @functools.partial(
        pl.kernel,
        out_type=jax.ShapeDtypeStruct((n, D), dt),
        mesh=mesh,
        scratch_types=[
            pltpu.VMEM((rows_w,), jnp.int32),
            pltpu.VMEM((rows_w, D), dt),
            pltpu.SemaphoreType.DMA,
        ],
    )
    def k(table_hbm, idx_hbm, out_hbm, idx_v, rows_v, sem):
        wid = lax.axis_index("s") * NC + lax.axis_index("c")
        base = wid * rows_w
        pltpu.sync_copy(idx_hbm.at[pl.ds(base, rows_w)], idx_v)
        pltpu.async_copy(table_hbm.at[idx_v], rows_v, sem).wait()
        pltpu.sync_copy(rows_v, out_hbm.at[pl.ds(base, rows_w)])

    return k(table, ids)


def _gmm_body(te_ref, xs_ref, W_hbm, b_ref, out_ref, W_vmem, sem):
    i = pl.program_id(0)

    @pl.when(i == 0)
    def _():
        for e in range(E):
            @pl.when(te_ref[0, NT + E + e] > 0)
            def _():
                pltpu.make_async_copy(
                    W_hbm.at[e], W_vmem.at[e], sem.at[e]).start()

    te = te_ref[0, i]
    first = jnp.logical_or(i == 0, te != te_ref[0, jnp.maximum(i - 1, 0)])

    @pl.when(first)
    def _():
        pltpu.make_async_copy(W_hbm.at[te], W_vmem.at[te], sem.at[te]).wait()

    out_ref[...] = lax.dot_general(
        xs_ref[...], W_vmem[te],
        (((1,), (1,)), ((), ())),
        preferred_element_type=jnp.float32,
    ) + b_ref[te]


def _grouped_matmul(x_sorted, expert_W, expert_b, te):
    grid_spec = pltpu.PrefetchScalarGridSpec(
        num_scalar_prefetch=1,
        grid=(NT,),
        in_specs=[
            pl.BlockSpec((TM, D), lambda i, te: (i, 0)),
            pl.BlockSpec(memory_space=pltpu.MemorySpace.HBM),
            pl.BlockSpec((E, 1, D), lambda i, te: (0, 0, 0)),
        ],
        out_specs=pl.BlockSpec((TM, D), lambda i, te: (i, 0)),
        scratch_shapes=[
            pltpu.VMEM((E, D, D), jnp.float32),
            pltpu.SemaphoreType.DMA((E,)),
        ],
    )
    return pl.pallas_call(
        _gmm_body,
        grid_spec=grid_spec,
        out_shape=jax.ShapeDtypeStruct((P, D), jnp.float32),
        compiler_params=pltpu.CompilerParams(
            dimension_semantics=("parallel",),
        ),
    )(te, x_sorted, expert_W, expert_b.reshape(E, 1, D))


def kernel(x, expert_W, expert_b, gate_W, gate_b):
    sid, dest, te = _routing(x, gate_W, gate_b)
    sids = sid[:NT].reshape(P)
    x_sorted = _sc_gather(x, sids)
    out_sorted = _grouped_matmul(x_sorted, expert_W, expert_b, te)
    return _sc_gather(out_sorted, dest.reshape(T))


# speedup vs baseline: 1.1596x; 1.0252x over previous
import functools

import jax
import jax.numpy as jnp
from jax import lax
from jax.experimental import pallas as pl
from jax.experimental.pallas import tpu as pltpu
from jax.experimental.pallas import tpu_sc as plsc

E = 8
T = 2048
D = 1024
LANES = 128
TM = 256
NT = 16
P = NT * TM
NC = 2
NS = 16
NW = NC * NS
CHUNK = 64


def _routing_body(x_ref, gW_ref, gb_ref, sid_ref, dest_ref, te_ref):
    f32 = jnp.float32
    i32 = jnp.int32
    x = x_ref[...]
    gate = jnp.dot(x, gW_ref[...], preferred_element_type=f32) + gb_ref[...]
    mx = jnp.max(gate, axis=1, keepdims=True)
    lane = lax.broadcasted_iota(i32, (T, LANES), 1)
    idx = jnp.min(jnp.where(gate == mx, lane, LANES), axis=1, keepdims=True)
    m = (lane == idx).astype(f32)

    CH = 256
    rch = lax.broadcasted_iota(i32, (CH, CH), 0)
    cch = lax.broadcasted_iota(i32, (CH, CH), 1)
    L = (rch >= cch).astype(f32)
    chunks = []
    carry = jnp.zeros((1, LANES), f32)
    for ci in range(T // CH):
        cs = jnp.dot(L, m[ci * CH:(ci + 1) * CH, :],
                     preferred_element_type=f32) + carry
        carry = cs[CH - 1:CH, :]
        chunks.append(cs)
    csum = jnp.concatenate(chunks, axis=0)

    rank = jnp.sum(csum * m, axis=1, keepdims=True) - 1.0
    counts_row = csum[T - 1:T, :]
    nt_row = (counts_row.astype(i32) + (TM - 1)) >> 8
    pc_row = (nt_row << 8).astype(f32)

    r2 = lax.broadcasted_iota(i32, (LANES, LANES), 0)
    c2 = lax.broadcasted_iota(i32, (LANES, LANES), 1)
    SU = (r2 < c2).astype(f32)
    base_row = jnp.dot(pc_row, SU, preferred_element_type=f32)
    base_tok = jnp.sum(base_row * m, axis=1, keepdims=True)
    dest = (base_tok + rank).astype(i32)
    dest_ref[...] = dest

    SL = (r2 > c2).astype(f32)
    counts_col = lax.dot_general(
        m, jnp.ones((T, 1), f32), (((0,), (0,)), ((), ())),
        preferred_element_type=f32)
    nt_col = (counts_col.astype(i32) + (TM - 1)) >> 8
    pc_col = (nt_col << 8).astype(f32)
    base_col = jnp.dot(SL, pc_col, preferred_element_type=f32).astype(i32)
    cmp = (base_col <= c2 * TM).astype(i32)
    te = jnp.clip(jnp.sum(cmp, axis=0, keepdims=True) - 1, 0, E - 1)
    total_tiles = jnp.sum(nt_row, axis=1, keepdims=True)
    iotaL = lax.broadcasted_iota(i32, (1, LANES), 1)
    te_last = jnp.max(jnp.where(counts_row > 0.0, iotaL, -1),
                      axis=1, keepdims=True)
    te_full = jnp.where(iotaL < total_tiles, te, te_last)
    SH = ((c2 == r2 + NT + E) & (r2 < E)).astype(f32)
    nt_shift = jnp.dot(nt_row.astype(f32), SH,
                       preferred_element_type=f32).astype(i32)
    te_ref[...] = jnp.where(iotaL < NT, te_full, nt_shift)

    q = dest >> 7
    rr = dest & (LANES - 1)
    OH = (rr == lane).astype(f32)
    tvec = lax.broadcasted_iota(i32, (T, 1), 0).astype(f32)
    Bmask = (q == lane).astype(f32)
    B = Bmask * tvec
    S = lax.dot_general(B, OH, (((0,), (0,)), ((), ())),
                        preferred_element_type=f32,
                        precision=lax.Precision.HIGHEST)
    V = lax.dot_general(Bmask, OH, (((0,), (0,)), ((), ())),
                        preferred_element_type=f32)
    pidx = (((r2 << 7) + c2) & (T - 1)).astype(f32)
    sid_ref[...] = (S + (1.0 - V) * pidx).astype(i32)


def _routing(x, gate_W, gate_b):
    gWp = jnp.zeros((D, LANES), jnp.float32).at[:, :E].set(gate_W)
    gbp = jnp.full((1, LANES), -1e30, jnp.float32).at[0, :E].set(gate_b)
    return pl.pallas_call(
        _routing_body,
        out_shape=[
            jax.ShapeDtypeStruct((LANES, LANES), jnp.int32),
            jax.ShapeDtypeStruct((T, 1), jnp.int32),
            jax.ShapeDtypeStruct((1, LANES), jnp.int32),
        ],
    )(x, gWp, gbp)


def _sc_gather(table, ids):
    n = ids.shape[0]
    rows_w = n // NW
    dt = table.dtype
    mesh = plsc.VectorSubcoreMesh(core_axis_name="c", subcore_axis_name="s")

    @functools.partial(
        pl.kernel,
        out_type=jax.ShapeDtypeStruct((n, D), dt),
        mesh=mesh,
        scratch_types=[
            pltpu.VMEM((CHUNK,), jnp.int32),
            pltpu.VMEM((CHUNK, D), dt),
            pltpu.SemaphoreType.DMA,
        ],
    )
    def k(table_hbm, idx_hbm, out_hbm, idx_v, rows_v, sem):
        wid = lax.axis_index("s") * NC + lax.axis_index("c")
        for c in range(rows_w // CHUNK):
            base = wid * rows_w + c * CHUNK
            pltpu.sync_copy(idx_hbm.at[pl.ds(base, CHUNK)], idx_v)
            pltpu.async_copy(table_hbm.at[idx_v], rows_v, sem).wait()
            pltpu.sync_copy(rows_v, out_hbm.at[pl.ds(base, CHUNK)])

    return k(table, ids)


def _gmm_body(te_ref, xs_ref, W_hbm, b_ref, out_ref, W_vmem, sem):
    i = pl.program_id(0)

    @pl.when(i == 0)
    def _():
        for e in range(E):
            @pl.when(te_ref[0, NT + E + e] > 0)
            def _():
                pltpu.make_async_copy(
                    W_hbm.at[e], W_vmem.at[e], sem.at[e]).start()

    te = te_ref[0, i]
    first = jnp.logical_or(i == 0, te != te_ref[0, jnp.maximum(i - 1, 0)])

    @pl.when(first)
    def _():
        pltpu.make_async_copy(W_hbm.at[te], W_vmem.at[te], sem.at[te]).wait()

    out_ref[...] = lax.dot_general(
        xs_ref[...], W_vmem[te],
        (((1,), (1,)), ((), ())),
        preferred_element_type=jnp.float32,
    ) + b_ref[te]


def _grouped_matmul(x_sorted, expert_W, expert_b, te):
    grid_spec = pltpu.PrefetchScalarGridSpec(
        num_scalar_prefetch=1,
        grid=(NT,),
        in_specs=[
            pl.BlockSpec((TM, D), lambda i, te: (i, 0)),
            pl.BlockSpec(memory_space=pltpu.MemorySpace.HBM),
            pl.BlockSpec((E, 1, D), lambda i, te: (0, 0, 0)),
        ],
        out_specs=pl.BlockSpec((TM, D), lambda i, te: (i, 0)),
        scratch_shapes=[
            pltpu.VMEM((E, D, D), jnp.float32),
            pltpu.SemaphoreType.DMA((E,)),
        ],
    )
    return pl.pallas_call(
        _gmm_body,
        grid_spec=grid_spec,
        out_shape=jax.ShapeDtypeStruct((P, D), jnp.float32),
        compiler_params=pltpu.CompilerParams(
            dimension_semantics=("parallel",),
        ),
    )(te, x_sorted, expert_W, expert_b.reshape(E, 1, D))


def kernel(x, expert_W, expert_b, gate_W, gate_b):
    sid, dest, te = _routing(x, gate_W, gate_b)
    sids = sid[:P // LANES].reshape(P)
    x_sorted = _sc_gather(x, sids)
    out_sorted = _grouped_matmul(x_sorted, expert_W, expert_b, te)
    return _sc_gather(out_sorted, dest.reshape(T))
